# Initial kernel scaffold; baseline (speedup 1.0000x reference)
import jax
import jax.numpy as jnp
from jax.experimental import pallas as pl


def kernel(x, Wc, bc, Wg, bg):
    raise NotImplementedError("write your pallas kernel here")


# speedup vs baseline: 1.1717x; 1.1717x over previous
import functools

import jax
import jax.numpy as jnp
from jax import lax
from jax.experimental import pallas as pl
from jax.experimental.pallas import tpu as pltpu

_DIMS = 4096
_T = 64
_K = 8
_ROWS = 256


def _body(x_ref, w_ref, b_ref, o_ref):
    z = lax.dot_general(
        x_ref[...], w_ref[...],
        dimension_numbers=(((1,), (1,)), ((), ())),
        preferred_element_type=jnp.float32,
    ) + b_ref[...]
    c = z[:, :_T]
    sig = 1.0 / (1.0 + jnp.exp(-z[:, _T:]))

    iota = lax.broadcasted_iota(jnp.int32, (_ROWS, _T), 1)
    neg = jnp.finfo(jnp.float32).min
    work = c
    m = jnp.max(work, axis=1, keepdims=True)
    num = jnp.zeros((_ROWS, 1), jnp.float32)
    den = jnp.zeros((_ROWS, 1), jnp.float32)
    for _ in range(_K):
        vk = jnp.max(work, axis=1, keepdims=True)
        eq = work == vk
        sel_idx = jnp.min(jnp.where(eq, iota, _T), axis=1, keepdims=True)
        sel = iota == sel_idx
        wk = jnp.exp(vk - m)
        gk = jnp.sum(jnp.where(sel, sig, 0.0), axis=1, keepdims=True)
        num = num + wk * gk
        den = den + wk
        work = jnp.where(sel, neg, work)
    o_ref[...] = num / den


def kernel(x, Wc, bc, Wg, bg):
    B, S, D = x.shape
    n = B * S
    xf = x.reshape(n, D)
    w = jnp.concatenate([Wc, Wg], axis=0)
    b = jnp.concatenate([bc, bg], axis=0)[None, :]
    out = pl.pallas_call(
        _body,
        grid=(n // _ROWS,),
        in_specs=[
            pl.BlockSpec((_ROWS, D), lambda i: (i, 0)),
            pl.BlockSpec((2 * _T, D), lambda i: (0, 0)),
            pl.BlockSpec((1, 2 * _T), lambda i: (0, 0)),
        ],
        out_specs=pl.BlockSpec((_ROWS, 1), lambda i: (i, 0)),
        out_shape=jax.ShapeDtypeStruct((n, 1), jnp.float32),
    )(xf, w, b)
    return out.reshape(B, S, 1)


# speedup vs baseline: 1.4375x; 1.2269x over previous
import functools

import jax
import jax.numpy as jnp
from jax import lax
from jax.experimental import pallas as pl
from jax.experimental.pallas import tpu as pltpu

_DIMS = 4096
_T = 64
_K = 8
_ROWS = 256


def _body(x_ref, w_ref, b_ref, o_ref):
    z = lax.dot_general(
        x_ref[...], w_ref[...],
        dimension_numbers=(((1,), (1,)), ((), ())),
        preferred_element_type=jnp.float32,
    ) + b_ref[...]
    c = z[:, :_T]
    sig = 1.0 / (1.0 + jnp.exp(-z[:, _T:]))

    bits = lax.bitcast_convert_type(c, jnp.int32)
    skey = jnp.where(bits >= 0, bits, bits ^ jnp.int32(0x7FFFFFFF))
    iota = lax.broadcasted_iota(jnp.int32, (_ROWS, _T), 1)
    key = (skey & jnp.int32(~63)) | (jnp.int32(_T - 1) - iota)

    imin = jnp.iinfo(jnp.int32).min
    topmask = jnp.zeros((_ROWS, _T), jnp.bool_)
    for _ in range(_K):
        kmax = jnp.max(key, axis=1, keepdims=True)
        sel = key == kmax
        topmask = topmask | sel
        key = jnp.where(sel, imin, key)

    m = jnp.max(c, axis=1, keepdims=True)
    e = jnp.where(topmask, jnp.exp(c - m), 0.0)
    num = jnp.sum(e * sig, axis=1, keepdims=True)
    den = jnp.sum(e, axis=1, keepdims=True)
    o_ref[...] = num / den


def kernel(x, Wc, bc, Wg, bg):
    B, S, D = x.shape
    n = B * S
    xf = x.reshape(n, D)
    w = jnp.concatenate([Wc, Wg], axis=0)
    b = jnp.concatenate([bc, bg], axis=0)[None, :]
    out = pl.pallas_call(
        _body,
        grid=(n // _ROWS,),
        in_specs=[
            pl.BlockSpec((_ROWS, D), lambda i: (i, 0)),
            pl.BlockSpec((2 * _T, D), lambda i: (0, 0)),
            pl.BlockSpec((1, 2 * _T), lambda i: (0, 0)),
        ],
        out_specs=pl.BlockSpec((_ROWS, 1), lambda i: (i, 0)),
        out_shape=jax.ShapeDtypeStruct((n, 1), jnp.float32),
    )(xf, w, b)
    return out.reshape(B, S, 1)


# speedup vs baseline: 2.1473x; 1.4937x over previous
import functools

import jax
import jax.numpy as jnp
from jax import lax
from jax.experimental import pallas as pl
from jax.experimental.pallas import tpu as pltpu

_DIMS = 4096
_T = 64
_K = 8
_ROWS = 256


def _body(x_ref, w_ref, b_ref, o_ref):
    z = lax.dot_general(
        x_ref[...], w_ref[...],
        dimension_numbers=(((1,), (1,)), ((), ())),
        preferred_element_type=jnp.float32,
    ) + b_ref[...]
    zt = z.T
    c = zt[:_T, :]
    sig = 1.0 / (1.0 + jnp.exp(-zt[_T:, :]))

    bits = lax.bitcast_convert_type(c, jnp.int32)
    skey = jnp.where(bits >= 0, bits, bits ^ jnp.int32(0x7FFFFFFF))
    iota = lax.broadcasted_iota(jnp.int32, (_T, _ROWS), 0)
    key = (skey & jnp.int32(~63)) | (jnp.int32(_T - 1) - iota)

    imin = jnp.iinfo(jnp.int32).min
    topmask = jnp.zeros((_T, _ROWS), jnp.bool_)
    for _ in range(_K):
        kmax = jnp.max(key, axis=0, keepdims=True)
        sel = key == kmax
        topmask = topmask | sel
        key = jnp.where(sel, imin, key)

    m = jnp.max(c, axis=0, keepdims=True)
    e = jnp.where(topmask, jnp.exp(c - m), 0.0)
    num = jnp.sum(e * sig, axis=0, keepdims=True)
    den = jnp.sum(e, axis=0, keepdims=True)
    o_ref[...] = num / den


def kernel(x, Wc, bc, Wg, bg):
    B, S, D = x.shape
    n = B * S
    xf = x.reshape(n, D)
    w = jnp.concatenate([Wc, Wg], axis=0)
    b = jnp.concatenate([bc, bg], axis=0)[None, :]
    out = pl.pallas_call(
        _body,
        grid=(n // _ROWS,),
        in_specs=[
            pl.BlockSpec((_ROWS, D), lambda i: (i, 0)),
            pl.BlockSpec((2 * _T, D), lambda i: (0, 0)),
            pl.BlockSpec((1, 2 * _T), lambda i: (0, 0)),
        ],
        out_specs=pl.BlockSpec((1, _ROWS), lambda i: (0, i)),
        out_shape=jax.ShapeDtypeStruct((1, n), jnp.float32),
    )(xf, w, b)
    return out.reshape(B, S, 1)


# speedup vs baseline: 2.5409x; 1.1833x over previous
import functools

import jax
import jax.numpy as jnp
from jax import lax
from jax.experimental import pallas as pl
from jax.experimental.pallas import tpu as pltpu

_DIMS = 4096
_T = 64
_K = 8
_ROWS = 512


def _body(x_ref, w_ref, b_ref, o_ref):
    z = lax.dot_general(
        x_ref[...], w_ref[...],
        dimension_numbers=(((1,), (1,)), ((), ())),
        preferred_element_type=jnp.float32,
    ) + b_ref[...]
    zt = z.T
    c = zt[:_T, :]
    sig = 1.0 / (1.0 + jnp.exp(-zt[_T:, :]))

    bits = lax.bitcast_convert_type(c, jnp.int32)
    skey = jnp.where(bits >= 0, bits, bits ^ jnp.int32(0x7FFFFFFF))
    iota = lax.broadcasted_iota(jnp.int32, (_T, _ROWS), 0)
    key = (skey & jnp.int32(~63)) | (jnp.int32(_T - 1) - iota)

    imin = jnp.iinfo(jnp.int32).min
    topmask = jnp.zeros((_T, _ROWS), jnp.bool_)
    for _ in range(_K):
        kmax = jnp.max(key, axis=0, keepdims=True)
        sel = key == kmax
        topmask = topmask | sel
        key = jnp.where(sel, imin, key)

    m = jnp.max(c, axis=0, keepdims=True)
    e = jnp.where(topmask, jnp.exp(c - m), 0.0)
    num = jnp.sum(e * sig, axis=0, keepdims=True)
    den = jnp.sum(e, axis=0, keepdims=True)
    o_ref[...] = num / den


def kernel(x, Wc, bc, Wg, bg):
    B, S, D = x.shape
    n = B * S
    xf = x.reshape(n, D)
    w = jnp.concatenate([Wc, Wg], axis=0)
    b = jnp.concatenate([bc, bg], axis=0)[None, :]
    out = pl.pallas_call(
        _body,
        grid=(n // _ROWS,),
        in_specs=[
            pl.BlockSpec((_ROWS, D), lambda i: (i, 0)),
            pl.BlockSpec((2 * _T, D), lambda i: (0, 0)),
            pl.BlockSpec((1, 2 * _T), lambda i: (0, 0)),
        ],
        out_specs=pl.BlockSpec((1, _ROWS), lambda i: (0, i)),
        out_shape=jax.ShapeDtypeStruct((1, n), jnp.float32),
    )(xf, w, b)
    return out.reshape(B, S, 1)


# speedup vs baseline: 2.5519x; 1.0043x over previous
import functools

import jax
import jax.numpy as jnp
from jax import lax
from jax.experimental import pallas as pl
from jax.experimental.pallas import tpu as pltpu

_DIMS = 4096
_T = 64
_K = 8
_ROWS = 1024


def _body(x_ref, w_ref, b_ref, o_ref):
    z = lax.dot_general(
        x_ref[...], w_ref[...],
        dimension_numbers=(((1,), (1,)), ((), ())),
        preferred_element_type=jnp.float32,
    ) + b_ref[...]
    zt = z.T
    c = zt[:_T, :]
    sig = 1.0 / (1.0 + jnp.exp(-zt[_T:, :]))

    bits = lax.bitcast_convert_type(c, jnp.int32)
    skey = jnp.where(bits >= 0, bits, bits ^ jnp.int32(0x7FFFFFFF))
    iota = lax.broadcasted_iota(jnp.int32, (_T, _ROWS), 0)
    key = (skey & jnp.int32(~63)) | (jnp.int32(_T - 1) - iota)

    imin = jnp.iinfo(jnp.int32).min
    topmask = jnp.zeros((_T, _ROWS), jnp.bool_)
    for _ in range(_K):
        kmax = jnp.max(key, axis=0, keepdims=True)
        sel = key == kmax
        topmask = topmask | sel
        key = jnp.where(sel, imin, key)

    m = jnp.max(c, axis=0, keepdims=True)
    e = jnp.where(topmask, jnp.exp(c - m), 0.0)
    num = jnp.sum(e * sig, axis=0, keepdims=True)
    den = jnp.sum(e, axis=0, keepdims=True)
    o_ref[...] = num / den


def kernel(x, Wc, bc, Wg, bg):
    B, S, D = x.shape
    n = B * S
    xf = x.reshape(n, D)
    w = jnp.concatenate([Wc, Wg], axis=0)
    b = jnp.concatenate([bc, bg], axis=0)[None, :]
    out = pl.pallas_call(
        _body,
        grid=(n // _ROWS,),
        in_specs=[
            pl.BlockSpec((_ROWS, D), lambda i: (i, 0)),
            pl.BlockSpec((2 * _T, D), lambda i: (0, 0)),
            pl.BlockSpec((1, 2 * _T), lambda i: (0, 0)),
        ],
        out_specs=pl.BlockSpec((1, _ROWS), lambda i: (0, i)),
        out_shape=jax.ShapeDtypeStruct((1, n), jnp.float32),
    )(xf, w, b)
    return out.reshape(B, S, 1)


# speedup vs baseline: 2.5627x; 1.0042x over previous
import functools

import jax
import jax.numpy as jnp
from jax import lax
from jax.experimental import pallas as pl
from jax.experimental.pallas import tpu as pltpu

_DIMS = 4096
_T = 64
_K = 8
_ROWS = 1024


def _body(x1_ref, x2_ref, w1_ref, w2_ref, b_ref, o_ref):
    z = lax.dot_general(
        x1_ref[...], w1_ref[...],
        dimension_numbers=(((1,), (1,)), ((), ())),
        preferred_element_type=jnp.float32,
    ) + lax.dot_general(
        x2_ref[...], w2_ref[...],
        dimension_numbers=(((1,), (1,)), ((), ())),
        preferred_element_type=jnp.float32,
    ) + b_ref[...]
    zt = z.T
    c = zt[:_T, :]
    sig = 1.0 / (1.0 + jnp.exp(-zt[_T:, :]))

    bits = lax.bitcast_convert_type(c, jnp.int32)
    skey = jnp.where(bits >= 0, bits, bits ^ jnp.int32(0x7FFFFFFF))
    iota = lax.broadcasted_iota(jnp.int32, (_T, _ROWS), 0)
    key = (skey & jnp.int32(~63)) | (jnp.int32(_T - 1) - iota)

    imin = jnp.iinfo(jnp.int32).min
    topmask = jnp.zeros((_T, _ROWS), jnp.bool_)
    for _ in range(_K):
        kmax = jnp.max(key, axis=0, keepdims=True)
        sel = key == kmax
        topmask = topmask | sel
        key = jnp.where(sel, imin, key)

    m = jnp.max(c, axis=0, keepdims=True)
    e = jnp.where(topmask, jnp.exp(c - m), 0.0)
    num = jnp.sum(e * sig, axis=0, keepdims=True)
    den = jnp.sum(e, axis=0, keepdims=True)
    o_ref[...] = num / den


def kernel(x, Wc, bc, Wg, bg):
    B, S, D = x.shape
    n = B * S
    xf = x.reshape(n, D)
    h = D // 2
    w = jnp.concatenate([Wc, Wg], axis=0)
    b = jnp.concatenate([bc, bg], axis=0)[None, :]
    out = pl.pallas_call(
        _body,
        grid=(n // _ROWS,),
        in_specs=[
            pl.BlockSpec((_ROWS, h), lambda i: (i, 0)),
            pl.BlockSpec((_ROWS, h), lambda i: (i, 1)),
            pl.BlockSpec((2 * _T, h), lambda i: (0, 0)),
            pl.BlockSpec((2 * _T, h), lambda i: (0, 1)),
            pl.BlockSpec((1, 2 * _T), lambda i: (0, 0)),
        ],
        out_specs=pl.BlockSpec((1, _ROWS), lambda i: (0, i)),
        out_shape=jax.ShapeDtypeStruct((1, n), jnp.float32),
    )(xf, xf, w, w, b)
    return out.reshape(B, S, 1)
